# Initial kernel scaffold; baseline (speedup 1.0000x reference)
import jax
import jax.numpy as jnp
from jax.experimental import pallas as pl


def kernel(m_idx, m_gs, node_in, node_tar, params):
    raise NotImplementedError("write your pallas kernel here")


# speedup vs baseline: 2.9806x; 2.9806x over previous
import functools

import jax
import jax.numpy as jnp
from jax import lax
from jax.experimental import pallas as pl
from jax.experimental.pallas import tpu as pltpu
from jax.experimental.pallas import tpu_sc as plsc

ND = 10000
NE = 320000
D = 128
NDP = 10240
NC, NS = 2, 16
NW = NC * NS
EPW = NE // NW
CH = 80
NCHUNK = EPW // CH
RPT = NDP // NS
ZR = 128
NR = 2000
EB = 3200
LANES = 16


def _sc_gather_body(xs_hbm, xd_hbm, src_hbm, dst_hbm, out_hbm,
                    si_v, di_v, a_v, b_v, sem_a, sem_b):
    wid = lax.axis_index("s") * NC + lax.axis_index("c")
    base = wid * EPW

    def chunk(i, carry):
        off = base + i * CH
        pltpu.sync_copy(src_hbm.at[pl.ds(off, CH)], si_v)
        pltpu.sync_copy(dst_hbm.at[pl.ds(off, CH)], di_v)
        cpa = pltpu.async_copy(xs_hbm.at[si_v], a_v, sem_a)
        cpb = pltpu.async_copy(xd_hbm.at[di_v], b_v, sem_b)
        cpa.wait()
        cpb.wait()

        def row(r, c2):
            for j in range(D // LANES):
                s = pl.ds(j * LANES, LANES)
                a_v[r, s] = jnp.maximum(a_v[r, s] + b_v[r, s], 0.0)
            return c2

        lax.fori_loop(0, CH, row, 0)
        pltpu.sync_copy(a_v, out_hbm.at[pl.ds(off, CH)])
        return carry

    lax.fori_loop(0, NCHUNK, chunk, 0)


def _sc_gather(xs, xd, src, dst):
    mesh = plsc.VectorSubcoreMesh(core_axis_name="c", subcore_axis_name="s")
    f = pl.kernel(
        _sc_gather_body,
        mesh=mesh,
        out_type=jax.ShapeDtypeStruct((NE, D), jnp.float32),
        scratch_types=[
            pltpu.VMEM((CH,), jnp.int32),
            pltpu.VMEM((CH,), jnp.int32),
            pltpu.VMEM((CH, D), jnp.float32),
            pltpu.VMEM((CH, D), jnp.float32),
            pltpu.SemaphoreType.DMA,
            pltpu.SemaphoreType.DMA,
        ],
    )
    return f(xs, xd, src, dst)


def _sc_scatter_body(msg_hbm, dst_hbm, out_hbm, di_v, m_v, z_v, acc_sh):
    cid = lax.axis_index("c")
    sid = lax.axis_index("s")
    wid = sid * NC + cid

    def zrow(r, c):
        for j in range(D // LANES):
            z_v[r, pl.ds(j * LANES, LANES)] = jnp.zeros((LANES,), jnp.float32)
        return c

    lax.fori_loop(0, ZR, zrow, 0)

    def zcopy(k, c):
        pltpu.sync_copy(z_v, acc_sh.at[pl.ds(sid * RPT + k * ZR, ZR)])
        return c

    lax.fori_loop(0, RPT // ZR, zcopy, 0)
    plsc.subcore_barrier()

    base = wid * EPW

    def chunk(i, carry):
        off = base + i * CH
        pltpu.sync_copy(dst_hbm.at[pl.ds(off, CH)], di_v)
        pltpu.sync_copy(msg_hbm.at[pl.ds(off, CH)], m_v)
        pltpu.sync_copy(m_v, acc_sh.at[di_v], add=True)
        return carry

    lax.fori_loop(0, NCHUNK, chunk, 0)
    plsc.subcore_barrier()

    def ocopy(k, c):
        r0 = sid * RPT + k * ZR
        pltpu.sync_copy(acc_sh.at[pl.ds(r0, ZR)], out_hbm.at[cid, pl.ds(r0, ZR)])
        return c

    lax.fori_loop(0, RPT // ZR, ocopy, 0)


def _sc_scatter(msg, dst):
    mesh = plsc.VectorSubcoreMesh(core_axis_name="c", subcore_axis_name="s")
    f = pl.kernel(
        _sc_scatter_body,
        mesh=mesh,
        out_type=jax.ShapeDtypeStruct((NC, NDP, D), jnp.float32),
        scratch_types=[
            pltpu.VMEM((CH,), jnp.int32),
            pltpu.VMEM((CH, D), jnp.float32),
            pltpu.VMEM((ZR, D), jnp.float32),
            pltpu.VMEM_SHARED((NDP, D), jnp.float32),
        ],
    )
    return f(msg, dst)


def _layernorm(v, g, b):
    mu = jnp.mean(v, axis=1, keepdims=True)
    var = jnp.mean((v - mu) ** 2, axis=1, keepdims=True)
    return (v - mu) * lax.rsqrt(var + 1e-5) * g + b


def _dot(a, b):
    return jnp.dot(a, b, preferred_element_type=jnp.float32)


def _pos_measure(nin, tar):
    t = nin[:, 4:5]
    measure = jnp.logical_or(t == 0.0, t == 5.0)
    pos = jnp.where(measure, nin[:, 0:2], tar[:, 0:2])
    return t, measure, pos


def _encode_body(nin_ref, tar_ref, w0_ref, b0_ref, w1_ref, b1_ref, w2_ref,
                 b2_ref, g_ref, be_ref, ws_ref, wd_ref, bse_ref,
                 x_ref, xs_ref, xd_ref):
    nin = nin_ref[...]
    tar = tar_ref[...]
    t, _, pos = _pos_measure(nin, tar)
    w0 = w0_ref[...]
    h = (pos[:, 0:1] * w0[0:1, :] + pos[:, 1:2] * w0[1:2, :]
         + t * w0[2:3, :] + b0_ref[...])
    h = jnp.maximum(h, 0.0)
    h = jnp.maximum(_dot(h, w1_ref[...]) + b1_ref[...], 0.0)
    x = _layernorm(_dot(h, w2_ref[...]) + b2_ref[...], g_ref[...], be_ref[...])
    x_ref[...] = x
    xs_ref[...] = _dot(x, ws_ref[...]) + bse_ref[...]
    xd_ref[...] = _dot(x, wd_ref[...]) + bse_ref[...]


def _edge_body(h_ref, w1_ref, b1_ref, w2_ref, b2_ref, g_ref, be_ref, out_ref):
    h = jnp.maximum(_dot(h_ref[...], w1_ref[...]) + b1_ref[...], 0.0)
    out_ref[...] = _layernorm(_dot(h, w2_ref[...]) + b2_ref[...],
                              g_ref[...], be_ref[...])


def _node_body(x_ref, a0_ref, a1_ref, w0x_ref, w0a_ref, b0_ref, w1_ref,
               b1_ref, w2_ref, b2_ref, g_ref, be_ref, ws_ref, wd_ref, bse_ref,
               xn_ref, xs_ref, xd_ref):
    x = x_ref[...]
    agg = a0_ref[...] + a1_ref[...]
    h = jnp.maximum(_dot(x, w0x_ref[...]) + _dot(agg, w0a_ref[...])
                    + b0_ref[...], 0.0)
    h = jnp.maximum(_dot(h, w1_ref[...]) + b1_ref[...], 0.0)
    u = _layernorm(_dot(h, w2_ref[...]) + b2_ref[...], g_ref[...], be_ref[...])
    xn = x + u
    xn_ref[...] = xn
    xs_ref[...] = _dot(xn, ws_ref[...]) + bse_ref[...]
    xd_ref[...] = _dot(xn, wd_ref[...]) + bse_ref[...]


def _decode_body(x_ref, nin_ref, tar_ref, w0_ref, b0_ref, w1_ref, b1_ref,
                 w2t_ref, b2_ref, out_ref, loss_ref, nz_ref):
    i = pl.program_id(0)
    nin = nin_ref[...]
    tar = tar_ref[...]
    _, measure, pos = _pos_measure(nin, tar)
    h = jnp.maximum(_dot(x_ref[...], w0_ref[...]) + b0_ref[...], 0.0)
    h = jnp.maximum(_dot(h, w1_ref[...]) + b1_ref[...], 0.0)
    w2t = w2t_ref[...]
    o0 = jnp.sum(h * w2t[0:1, :], axis=1, keepdims=True) + b2_ref[0, 0]
    o1 = jnp.sum(h * w2t[1:2, :], axis=1, keepdims=True) + b2_ref[0, 1]
    o = jnp.concatenate([o0, o1], axis=1) + pos
    tar2 = tar[:, 0:2]
    fin = jnp.where(measure, o, tar2)
    out_ref[...] = fin
    se = jnp.sum(jnp.where(measure, (o - tar2) ** 2, 0.0))
    nzp = 2.0 * jnp.sum(measure.astype(jnp.float32))

    @pl.when(i == 0)
    def _():
        loss_ref[0, 0] = 0.0
        nz_ref[0, 0] = 0.0

    loss_ref[0, 0] += se
    nz_ref[0, 0] += nzp


def _full(shape):
    return pl.BlockSpec(shape, lambda i: (0, 0))


def _rows(shape):
    return pl.BlockSpec(shape, lambda i: (i, 0))


def _encode_call(nin8, tar8, w0p, b0, w1, b1, w2, b2, g, be, ws, wd, bse):
    grid = (ND // NR,)
    out = jax.ShapeDtypeStruct((ND, D), jnp.float32)
    return pl.pallas_call(
        _encode_body,
        grid=grid,
        in_specs=[_rows((NR, 8)), _rows((NR, 8)), _full((8, D)), _full((1, D)),
                  _full((D, D)), _full((1, D)), _full((D, D)), _full((1, D)),
                  _full((1, D)), _full((1, D)), _full((D, D)), _full((D, D)),
                  _full((1, D))],
        out_specs=[_rows((NR, D))] * 3,
        out_shape=[out] * 3,
    )(nin8, tar8, w0p, b0, w1, b1, w2, b2, g, be, ws, wd, bse)


def _edge_call(h1, w1, b1, w2, b2, g, be):
    grid = (NE // EB,)
    return pl.pallas_call(
        _edge_body,
        grid=grid,
        in_specs=[_rows((EB, D)), _full((D, D)), _full((1, D)), _full((D, D)),
                  _full((1, D)), _full((1, D)), _full((1, D))],
        out_specs=_rows((EB, D)),
        out_shape=jax.ShapeDtypeStruct((NE, D), jnp.float32),
    )(h1, w1, b1, w2, b2, g, be)


def _node_call(x, a0, a1, w0x, w0a, b0, w1, b1, w2, b2, g, be, ws, wd, bse):
    grid = (ND // NR,)
    out = jax.ShapeDtypeStruct((ND, D), jnp.float32)
    return pl.pallas_call(
        _node_body,
        grid=grid,
        in_specs=[_rows((NR, D))] * 3 + [
            _full((D, D)), _full((D, D)), _full((1, D)), _full((D, D)),
            _full((1, D)), _full((D, D)), _full((1, D)), _full((1, D)),
            _full((1, D)), _full((D, D)), _full((D, D)), _full((1, D))],
        out_specs=[_rows((NR, D))] * 3,
        out_shape=[out] * 3,
    )(x, a0, a1, w0x, w0a, b0, w1, b1, w2, b2, g, be, ws, wd, bse)


def _decode_call(x, nin8, tar8, w0, b0, w1, b1, w2t, b2):
    grid = (ND // NR,)
    scal = pl.BlockSpec((1, 1), lambda i: (0, 0), memory_space=pltpu.SMEM)
    return pl.pallas_call(
        _decode_body,
        grid=grid,
        in_specs=[_rows((NR, D)), _rows((NR, 8)), _rows((NR, 8)),
                  _full((D, D)), _full((1, D)), _full((D, D)), _full((1, D)),
                  _full((8, D)), _full((1, 8))],
        out_specs=[_rows((NR, 2)), scal, scal],
        out_shape=[jax.ShapeDtypeStruct((ND, 2), jnp.float32),
                   jax.ShapeDtypeStruct((1, 1), jnp.float32),
                   jax.ShapeDtypeStruct((1, 1), jnp.float32)],
    )(x, nin8, tar8, w0, b0, w1, b1, w2t, b2)


def _edge_parts(p):
    (w0, b0), (w1, b1), (w2, b2) = p["layers"]
    g, be = p["ln"]
    return dict(ws=w0[:D], wd=w0[D:], bse=(0.5 * b0).reshape(1, D),
                w1=w1, b1=b1.reshape(1, D), w2=w2, b2=b2.reshape(1, D),
                g=g.reshape(1, D), be=be.reshape(1, D))


def _node_parts(p):
    (w0, b0), (w1, b1), (w2, b2) = p["layers"]
    g, be = p["ln"]
    return dict(w0x=w0[:D], w0a=w0[D:], b0=b0.reshape(1, D),
                w1=w1, b1=b1.reshape(1, D), w2=w2, b2=b2.reshape(1, D),
                g=g.reshape(1, D), be=be.reshape(1, D))


def kernel(m_idx, m_gs, node_in, node_tar, params):
    nin = node_in[0]
    tar = node_tar[0]
    nin8 = jnp.pad(nin, ((0, 0), (0, 3)))
    tar8 = jnp.pad(tar, ((0, 0), (0, 6)))
    src = m_gs[0].astype(jnp.int32)
    dst = m_gs[1].astype(jnp.int32)

    enc = params["encode"]
    (ew0, eb0), (ew1, eb1), (ew2, eb2) = enc["layers"]
    eg, ebe = enc["ln"]
    ew0p = jnp.pad(ew0, ((0, 5), (0, 0)))

    edge_p = [_edge_parts(params["gn"][i]["edge"]) for i in range(2)]
    node_p = [_node_parts(params["gn"][i]["node"]) for i in range(2)]

    e0 = edge_p[0]
    x, xs, xd = _encode_call(nin8, tar8, ew0p, eb0.reshape(1, D), ew1,
                             eb1.reshape(1, D), ew2, eb2.reshape(1, D),
                             eg.reshape(1, D), ebe.reshape(1, D),
                             e0["ws"], e0["wd"], e0["bse"])

    order = [0, 1, 0, 1]
    for r in range(4):
        ep = edge_p[order[r]]
        np_ = node_p[order[r]]
        nxt = edge_p[order[r + 1]] if r < 3 else edge_p[0]
        h1 = _sc_gather(xs, xd, src, dst)
        msg = _edge_call(h1, ep["w1"], ep["b1"], ep["w2"], ep["b2"],
                         ep["g"], ep["be"])
        agg2 = _sc_scatter(msg, dst)
        a0 = agg2[0, :ND]
        a1 = agg2[1, :ND]
        x, xs, xd = _node_call(x, a0, a1, np_["w0x"], np_["w0a"], np_["b0"],
                               np_["w1"], np_["b1"], np_["w2"], np_["b2"],
                               np_["g"], np_["be"], nxt["ws"], nxt["wd"],
                               nxt["bse"])

    dec = params["decode"]
    (dw0, db0), (dw1, db1), (dw2, db2) = dec["layers"]
    dw2t = jnp.pad(dw2.T, ((0, 6), (0, 0)))
    db2p = jnp.pad(db2, (0, 6)).reshape(1, 8)
    out2, loss_s, nz_s = _decode_call(x, nin8, tar8, dw0, db0.reshape(1, D),
                                      dw1, db1.reshape(1, D), dw2t, db2p)
    nz = nz_s.reshape(())
    loss = loss_s.reshape(()) / nz
    return (loss, out2[None], nz)
